# baseline (speedup 1.0000x reference)
import jax
import jax.numpy as jnp
from jax.experimental import pallas as pl
from jax.experimental.pallas import tpu as pltpu


def _body(x_ref, t_ref, out_ref, acc_ref, *, npb, nb_total, chans):
    b = pl.program_id(0)
    j = pl.program_id(1)

    @pl.when(j == 0)
    def _():
        acc_ref[...] = jnp.zeros_like(acc_ref)

    xb = x_ref[0]
    tb = t_ref[0]
    pb = tb.shape[-1]
    lv = jax.lax.broadcasted_iota(jnp.int32, (16, pb), 0) + 1
    m16 = (tb == lv).astype(jnp.float32)
    dims = (((1,), (1,)), ((), ()))
    g1 = jax.lax.dot_general(m16, xb, dims,
                             preferred_element_type=jnp.float32)
    g2 = jax.lax.dot_general(m16, xb * xb, dims,
                             preferred_element_type=jnp.float32)
    ones_row = jnp.ones((1, pb), jnp.float32)
    n16 = jax.lax.dot_general(m16, ones_row, dims,
                              preferred_element_type=jnp.float32)
    acc_ref[0, :] += n16[:, 0]
    acc_ref[1, :] += jnp.sum(g1, axis=1)
    acc_ref[2, :] += jnp.sum(g2, axis=1)

    @pl.when(j == npb - 1)
    def _():
        n = acc_ref[0, :]
        S1 = acc_ref[1, :]
        S2 = acc_ref[2, :]
        m = jnp.float32(chans) * n
        var = (S2 - S1 * S1 / jnp.maximum(m, 1.0)) / jnp.maximum(m - 1.0, 1.0)
        var = jnp.where(n > 1.5, var, 0.0)
        cnt = jnp.sum((n > 0.5).astype(jnp.float32))
        loss_k = jnp.sum(var) / (cnt + jnp.float32(1e-8))

        @pl.when(b == 0)
        def _():
            out_ref[...] = jnp.zeros((1, 1), jnp.float32)

        out_ref[...] = out_ref[...] + (loss_k / jnp.float32(nb_total)).reshape(1, 1)


def kernel(input, target):
    B, C = input.shape[0], input.shape[1]
    P = input.shape[2] * input.shape[3]
    PB = min(P, 8192)
    NPB = P // PB
    x = input.reshape(B, C, P)
    t = target.reshape(B, 1, P).astype(jnp.int32)

    import functools
    body = functools.partial(_body, npb=NPB, nb_total=B, chans=C)
    out = pl.pallas_call(
        body,
        grid=(B, NPB),
        in_specs=[
            pl.BlockSpec((1, C, PB), lambda b, j: (b, 0, j)),
            pl.BlockSpec((1, 1, PB), lambda b, j: (b, 0, j)),
        ],
        out_specs=pl.BlockSpec((1, 1), lambda b, j: (0, 0)),
        out_shape=jax.ShapeDtypeStruct((1, 1), jnp.float32),
        scratch_shapes=[pltpu.VMEM((3, 16), jnp.float32)],
    )(x, t)
    return out[0, 0]


# speedup vs baseline: 1.2170x; 1.2170x over previous
import jax
import jax.numpy as jnp
from jax.experimental import pallas as pl
from jax.experimental.pallas import tpu as pltpu


def _body(x_ref, t_ref, out_ref, acc_ref, *, npb, nb_total, chans):
    b = pl.program_id(0)
    j = pl.program_id(1)

    @pl.when(j == 0)
    def _():
        acc_ref[...] = jnp.zeros_like(acc_ref)

    xb = x_ref[0]
    tb = t_ref[0]
    pb = tb.shape[-1]
    lv = jax.lax.broadcasted_iota(jnp.int32, (16, pb), 0) + 1
    m16 = (tb == lv).astype(jnp.float32)
    dims = (((1,), (1,)), ((), ()))
    g1 = jax.lax.dot_general(m16, xb, dims,
                             preferred_element_type=jnp.float32)
    g2 = jax.lax.dot_general(m16, xb * xb, dims,
                             preferred_element_type=jnp.float32)
    ones_row = jnp.ones((1, pb), jnp.float32)
    n16 = jax.lax.dot_general(m16, ones_row, dims,
                              preferred_element_type=jnp.float32)
    acc_ref[0, :] += n16[:, 0]
    acc_ref[1, :] += jnp.sum(g1, axis=1)
    acc_ref[2, :] += jnp.sum(g2, axis=1)

    @pl.when(j == npb - 1)
    def _():
        n = acc_ref[0, :]
        S1 = acc_ref[1, :]
        S2 = acc_ref[2, :]
        m = jnp.float32(chans) * n
        var = (S2 - S1 * S1 / jnp.maximum(m, 1.0)) / jnp.maximum(m - 1.0, 1.0)
        var = jnp.where(n > 1.5, var, 0.0)
        cnt = jnp.sum((n > 0.5).astype(jnp.float32))
        loss_k = jnp.sum(var) / (cnt + jnp.float32(1e-8))

        @pl.when(b == 0)
        def _():
            out_ref[...] = jnp.zeros((1, 1), jnp.float32)

        out_ref[...] = out_ref[...] + (loss_k / jnp.float32(nb_total)).reshape(1, 1)


def kernel(input, target):
    B, C = input.shape[0], input.shape[1]
    P = input.shape[2] * input.shape[3]
    PB = min(P, 32768)
    NPB = P // PB
    x = input.reshape(B, C, P)
    t = target.reshape(B, 1, P).astype(jnp.int32)

    import functools
    body = functools.partial(_body, npb=NPB, nb_total=B, chans=C)
    out = pl.pallas_call(
        body,
        grid=(B, NPB),
        in_specs=[
            pl.BlockSpec((1, C, PB), lambda b, j: (b, 0, j)),
            pl.BlockSpec((1, 1, PB), lambda b, j: (b, 0, j)),
        ],
        out_specs=pl.BlockSpec((1, 1), lambda b, j: (0, 0)),
        out_shape=jax.ShapeDtypeStruct((1, 1), jnp.float32),
        scratch_shapes=[pltpu.VMEM((3, 16), jnp.float32)],
    )(x, t)
    return out[0, 0]
